# baseline (speedup 1.0000x reference)
import jax
import jax.numpy as jnp
from jax.experimental import pallas as pl
from jax.experimental.pallas import tpu as pltpu

_B = 8
_N = 8192
_NC = 512
_EPS = 1e-5


def _fps_body(xyz_ref, far0_ref, idx_ref, nx0_ref, nx1_ref, nx2_ref, dist_ref):
    x0 = xyz_ref[:, 0, :]
    x1 = xyz_ref[:, 1, :]
    x2 = xyz_ref[:, 2, :]
    x24 = jnp.concatenate([x0, x1, x2], axis=0)
    lane = jax.lax.broadcasted_iota(jnp.int32, (_B, _N), 1)
    col = jax.lax.broadcasted_iota(jnp.int32, (_B, _NC), 1)
    dmask = (jax.lax.broadcasted_iota(jnp.int32, (3 * _B, _B), 1)
             == jax.lax.broadcasted_iota(jnp.int32, (3 * _B, _B), 0) % _B)
    dist_ref[...] = jnp.full((_B, _N), 1e10, jnp.float32)
    idx_ref[...] = jnp.zeros((_B, _NC), jnp.int32)
    nx0_ref[...] = jnp.zeros((_B, _NC), jnp.float32)
    nx1_ref[...] = jnp.zeros((_B, _NC), jnp.float32)
    nx2_ref[...] = jnp.zeros((_B, _NC), jnp.float32)

    def body(i, far):
        onehot = jnp.where(lane == far, 1.0, 0.0)
        cmat = jax.lax.dot_general(x24, onehot, (((1,), (1,)), ((), ())),
                                   preferred_element_type=jnp.float32)
        cdiag = jnp.sum(jnp.where(dmask, cmat, 0.0), axis=1, keepdims=True)
        c0 = cdiag[0:_B]
        c1 = cdiag[_B:2 * _B]
        c2 = cdiag[2 * _B:3 * _B]
        d0 = x0 - c0
        d1 = x1 - c1
        d2 = x2 - c2
        d = d0 * d0 + d1 * d1 + d2 * d2
        dist = jnp.minimum(dist_ref[...], d)
        dist_ref[...] = dist
        hit = col == i
        idx_ref[...] = jnp.where(hit, jnp.broadcast_to(far, (_B, _NC)), idx_ref[...])
        nx0_ref[...] = jnp.where(hit, jnp.broadcast_to(c0, (_B, _NC)), nx0_ref[...])
        nx1_ref[...] = jnp.where(hit, jnp.broadcast_to(c1, (_B, _NC)), nx1_ref[...])
        nx2_ref[...] = jnp.where(hit, jnp.broadcast_to(c2, (_B, _NC)), nx2_ref[...])
        m = jnp.max(dist, axis=1, keepdims=True)
        far_new = jnp.min(jnp.where(dist == m, lane, _N), axis=1, keepdims=True)
        return far_new

    jax.lax.fori_loop(0, _NC, body, far0_ref[...])


def _fps_call(xyz, far0):
    return pl.pallas_call(
        _fps_body,
        out_shape=(
            jax.ShapeDtypeStruct((_B, _NC), jnp.int32),
            jax.ShapeDtypeStruct((_B, _NC), jnp.float32),
            jax.ShapeDtypeStruct((_B, _NC), jnp.float32),
            jax.ShapeDtypeStruct((_B, _NC), jnp.float32),
        ),
        scratch_shapes=[pltpu.VMEM((_B, _N), jnp.float32)],
    )(xyz, far0)


def _bn(h, g, be):
    m = jnp.mean(h, axis=1, keepdims=True)
    v = jnp.mean((h - m) ** 2, axis=1, keepdims=True)
    return (h - m) / jnp.sqrt(v + _EPS) * g + be


def _lrelu(h):
    return jnp.where(h >= 0, h, 0.2 * h)


def _mlp_body(xt_ref, w1_ref, b1_ref, g1_ref, be1_ref, w2_ref, b2_ref, g2_ref,
              be2_ref, w3_ref, b3_ref, f_ref):
    xt = xt_ref[...]
    h = jax.lax.dot_general(w1_ref[...], xt, (((1,), (0,)), ((), ())),
                            preferred_element_type=jnp.float32) + b1_ref[...]
    h = _lrelu(_bn(h, g1_ref[...], be1_ref[...]))
    h = jax.lax.dot_general(w2_ref[...], h, (((1,), (0,)), ((), ())),
                            preferred_element_type=jnp.float32) + b2_ref[...]
    h = _lrelu(_bn(h, g2_ref[...], be2_ref[...]))
    f_ref[...] = jax.lax.dot_general(w3_ref[...], h, (((1,), (0,)), ((), ())),
                                     preferred_element_type=jnp.float32) + b3_ref[...]


def _mlp_call(xt, w1, b1, g1, be1, w2, b2, g2, be2, w3, b3):
    return pl.pallas_call(
        _mlp_body,
        out_shape=jax.ShapeDtypeStruct((64, _B * _N), jnp.float32),
    )(xt, w1, b1, g1, be1, w2, b2, g2, be2, w3, b3)


def _attn_body(f_ref, idx_ref, wq_ref, wk_ref, wv_ref, wo_ref, out_ref):
    fb = f_ref[...]
    idxb = idx_ref[0]
    lane = jax.lax.broadcasted_iota(jnp.int32, (_NC, _N), 1)
    onehot = jnp.where(lane == idxb, 1.0, 0.0)
    cent = jax.lax.dot_general(onehot, fb, (((1,), (1,)), ((), ())),
                               preferred_element_type=jnp.float32)
    q = jax.lax.dot_general(cent, wq_ref[...], (((1,), (1,)), ((), ())),
                            preferred_element_type=jnp.float32)
    k = jax.lax.dot_general(wk_ref[...], fb, (((1,), (0,)), ((), ())),
                            preferred_element_type=jnp.float32)
    logits = jax.lax.dot_general(q, k, (((1,), (0,)), ((), ())),
                                 preferred_element_type=jnp.float32) * 0.125
    mx = jnp.max(logits, axis=1, keepdims=True)
    e = jnp.exp(logits - mx)
    probs = e / jnp.sum(e, axis=1, keepdims=True)
    t = jax.lax.dot_general(probs, fb, (((1,), (1,)), ((), ())),
                            preferred_element_type=jnp.float32)
    o = jax.lax.dot_general(t, wv_ref[...], (((1,), (1,)), ((), ())),
                            preferred_element_type=jnp.float32)
    y = jax.lax.dot_general(o, wo_ref[...], (((1,), (1,)), ((), ())),
                            preferred_element_type=jnp.float32)
    out_ref[0] = cent + y


def _attn_call(f, idx3, wq, wk, wv, wo):
    wspec = pl.BlockSpec((64, 64), lambda b: (0, 0))
    return pl.pallas_call(
        _attn_body,
        grid=(_B,),
        in_specs=[
            pl.BlockSpec((64, _N), lambda b: (0, b)),
            pl.BlockSpec((1, _NC, 1), lambda b: (b, 0, 0)),
            wspec, wspec, wspec, wspec,
        ],
        out_specs=pl.BlockSpec((1, _NC, 64), lambda b: (b, 0, 0)),
        out_shape=jax.ShapeDtypeStruct((_B, _NC, 64), jnp.float32),
        compiler_params=pltpu.CompilerParams(
            dimension_semantics=("arbitrary",)),
    )(f, idx3, wq, wk, wv, wo)


def kernel(xyz, W1, b1, g1, be1, W2, b2, g2, be2, W3, b3, Wq, Wk, Wv, Wo):
    far0 = jax.random.randint(jax.random.key(42), (_B,), 0, _N,
                              dtype=jnp.int32).reshape(_B, 1)
    idx, nx0, nx1, nx2 = _fps_call(xyz, far0)
    new_xyz = jnp.stack([nx0, nx1, nx2], axis=1)

    xt = xyz.transpose(1, 0, 2).reshape(3, _B * _N)
    col = lambda a: a.reshape(-1, 1)
    f = _mlp_call(xt, W1, col(b1), col(g1), col(be1), W2, col(b2), col(g2),
                  col(be2), W3, col(b3))

    out2t = _attn_call(f, idx.reshape(_B, _NC, 1), Wq, Wk, Wv, Wo)
    return (new_xyz, out2t.transpose(0, 2, 1))


# speedup vs baseline: 1.2466x; 1.2466x over previous
import jax
import jax.numpy as jnp
from jax.experimental import pallas as pl
from jax.experimental.pallas import tpu as pltpu

_B = 8
_N = 8192
_NC = 512
_EPS = 1e-5


def _fps_body(xyz_ref, far0_ref, idx_ref, nx0_ref, nx1_ref, nx2_ref, dist_ref):
    x0 = xyz_ref[:, 0, :]
    x1 = xyz_ref[:, 1, :]
    x2 = xyz_ref[:, 2, :]
    x24 = jnp.concatenate([x0, x1, x2], axis=0)
    lane = jax.lax.broadcasted_iota(jnp.int32, (_B, _N), 1)
    lane24 = jax.lax.broadcasted_iota(jnp.int32, (3 * _B, _N), 1)
    col = jax.lax.broadcasted_iota(jnp.int32, (_B, _NC), 1)
    dist_ref[...] = jnp.full((_B, _N), 1e10, jnp.float32)
    idx_ref[...] = jnp.zeros((_B, _NC), jnp.int32)
    nx0_ref[...] = jnp.zeros((_B, _NC), jnp.float32)
    nx1_ref[...] = jnp.zeros((_B, _NC), jnp.float32)
    nx2_ref[...] = jnp.zeros((_B, _NC), jnp.float32)

    def body(i, far):
        far24 = jnp.concatenate([far, far, far], axis=0)
        sel = lane24 == far24
        g = jnp.sum(jnp.where(sel, x24, 0.0), axis=1, keepdims=True)
        c0 = g[0:_B]
        c1 = g[_B:2 * _B]
        c2 = g[2 * _B:3 * _B]
        d0 = x0 - c0
        d1 = x1 - c1
        d2 = x2 - c2
        d = d0 * d0 + d1 * d1 + d2 * d2
        dist = jnp.minimum(dist_ref[...], d)
        dist_ref[...] = dist
        hit = col == i
        idx_ref[...] = jnp.where(hit, jnp.broadcast_to(far, (_B, _NC)), idx_ref[...])
        nx0_ref[...] = jnp.where(hit, jnp.broadcast_to(c0, (_B, _NC)), nx0_ref[...])
        nx1_ref[...] = jnp.where(hit, jnp.broadcast_to(c1, (_B, _NC)), nx1_ref[...])
        nx2_ref[...] = jnp.where(hit, jnp.broadcast_to(c2, (_B, _NC)), nx2_ref[...])
        m = jnp.max(dist, axis=1, keepdims=True)
        far_new = jnp.min(jnp.where(dist == m, lane, _N), axis=1, keepdims=True)
        return far_new

    jax.lax.fori_loop(0, _NC, body, far0_ref[...])


def _fps_call(xyz, far0):
    return pl.pallas_call(
        _fps_body,
        out_shape=(
            jax.ShapeDtypeStruct((_B, _NC), jnp.int32),
            jax.ShapeDtypeStruct((_B, _NC), jnp.float32),
            jax.ShapeDtypeStruct((_B, _NC), jnp.float32),
            jax.ShapeDtypeStruct((_B, _NC), jnp.float32),
        ),
        scratch_shapes=[pltpu.VMEM((_B, _N), jnp.float32)],
    )(xyz, far0)


def _bn(h, g, be):
    m = jnp.mean(h, axis=1, keepdims=True)
    v = jnp.mean((h - m) ** 2, axis=1, keepdims=True)
    return (h - m) / jnp.sqrt(v + _EPS) * g + be


def _lrelu(h):
    return jnp.where(h >= 0, h, 0.2 * h)


def _net_body(xt_ref, idx_ref, w1_ref, b1_ref, g1_ref, be1_ref, w2_ref, b2_ref,
              g2_ref, be2_ref, w3_ref, b3_ref, wq_ref, wk_ref, wv_ref, wo_ref,
              out_ref, f_scr):
    b = pl.program_id(0)

    @pl.when(b == 0)
    def _mlp():
        xt = xt_ref[...]
        h = jax.lax.dot_general(w1_ref[...], xt, (((1,), (0,)), ((), ())),
                                preferred_element_type=jnp.float32) + b1_ref[...]
        h = _lrelu(_bn(h, g1_ref[...], be1_ref[...]))
        h = jax.lax.dot_general(w2_ref[...], h, (((1,), (0,)), ((), ())),
                                preferred_element_type=jnp.float32) + b2_ref[...]
        h = _lrelu(_bn(h, g2_ref[...], be2_ref[...]))
        f_scr[...] = jax.lax.dot_general(w3_ref[...], h, (((1,), (0,)), ((), ())),
                                         preferred_element_type=jnp.float32) + b3_ref[...]

    fb = f_scr[:, pl.ds(b * _N, _N)]
    idxb = idx_ref[0]
    lane = jax.lax.broadcasted_iota(jnp.int32, (_NC, _N), 1)
    onehot = jnp.where(lane == idxb, 1.0, 0.0)
    cent = jax.lax.dot_general(onehot, fb, (((1,), (1,)), ((), ())),
                               preferred_element_type=jnp.float32)
    q = jax.lax.dot_general(cent, wq_ref[...], (((1,), (1,)), ((), ())),
                            preferred_element_type=jnp.float32)
    k = jax.lax.dot_general(wk_ref[...], fb, (((1,), (0,)), ((), ())),
                            preferred_element_type=jnp.float32)
    logits = jax.lax.dot_general(q, k, (((1,), (0,)), ((), ())),
                                 preferred_element_type=jnp.float32) * 0.125
    mx = jnp.max(logits, axis=1, keepdims=True)
    e = jnp.exp(logits - mx)
    probs = e / jnp.sum(e, axis=1, keepdims=True)
    t = jax.lax.dot_general(probs, fb, (((1,), (1,)), ((), ())),
                            preferred_element_type=jnp.float32)
    o = jax.lax.dot_general(t, wv_ref[...], (((1,), (1,)), ((), ())),
                            preferred_element_type=jnp.float32)
    y = jax.lax.dot_general(o, wo_ref[...], (((1,), (1,)), ((), ())),
                            preferred_element_type=jnp.float32)
    out_ref[0] = cent + y


def _net_call(xt, idx3, w1, b1, g1, be1, w2, b2, g2, be2, w3, b3, wq, wk, wv, wo):
    full2d = lambda a: pl.BlockSpec(a.shape, lambda b: (0,) * a.ndim)
    return pl.pallas_call(
        _net_body,
        grid=(_B,),
        in_specs=[
            full2d(xt),
            pl.BlockSpec((1, _NC, 1), lambda b: (b, 0, 0)),
        ] + [full2d(w) for w in (w1, b1, g1, be1, w2, b2, g2, be2, w3, b3,
                                 wq, wk, wv, wo)],
        out_specs=pl.BlockSpec((1, _NC, 64), lambda b: (b, 0, 0)),
        out_shape=jax.ShapeDtypeStruct((_B, _NC, 64), jnp.float32),
        scratch_shapes=[pltpu.VMEM((64, _B * _N), jnp.float32)],
        compiler_params=pltpu.CompilerParams(
            dimension_semantics=("arbitrary",)),
    )(xt, idx3, w1, b1, g1, be1, w2, b2, g2, be2, w3, b3, wq, wk, wv, wo)


def kernel(xyz, W1, b1, g1, be1, W2, b2, g2, be2, W3, b3, Wq, Wk, Wv, Wo):
    far0 = jax.random.randint(jax.random.key(42), (_B,), 0, _N,
                              dtype=jnp.int32).reshape(_B, 1)
    idx, nx0, nx1, nx2 = _fps_call(xyz, far0)
    new_xyz = jnp.stack([nx0, nx1, nx2], axis=1)

    xt = xyz.transpose(1, 0, 2).reshape(3, _B * _N)
    col = lambda a: a.reshape(-1, 1)
    out2t = _net_call(xt, idx.reshape(_B, _NC, 1), W1, col(b1), col(g1),
                      col(be1), W2, col(b2), col(g2), col(be2), W3, col(b3),
                      Wq, Wk, Wv, Wo)
    return (new_xyz, out2t.transpose(0, 2, 1))


# speedup vs baseline: 1.3187x; 1.0578x over previous
import jax
import jax.numpy as jnp
from jax.experimental import pallas as pl
from jax.experimental.pallas import tpu as pltpu

_B = 8
_N = 8192
_NC = 512
_EPS = 1e-5


def _fps_body(xyz_ref, far0_ref, idx_ref, nx0_ref, nx1_ref, nx2_ref, dist_ref):
    x0 = xyz_ref[:, 0, :]
    x1 = xyz_ref[:, 1, :]
    x2 = xyz_ref[:, 2, :]
    x24 = jnp.concatenate([x0, x1, x2], axis=0)
    lane = jax.lax.broadcasted_iota(jnp.int32, (_B, _N), 1)
    lane24 = jax.lax.broadcasted_iota(jnp.int32, (3 * _B, _N), 1)
    col = jax.lax.broadcasted_iota(jnp.int32, (_B, _NC), 1)
    dist_ref[...] = jnp.full((_B, _N), 1e10, jnp.float32)
    idx_ref[...] = jnp.zeros((_B, _NC), jnp.int32)
    nx0_ref[...] = jnp.zeros((_B, _NC), jnp.float32)
    nx1_ref[...] = jnp.zeros((_B, _NC), jnp.float32)
    nx2_ref[...] = jnp.zeros((_B, _NC), jnp.float32)

    def body(i, far):
        sel = lane == far
        sel24 = jnp.concatenate([sel, sel, sel], axis=0)
        g = jnp.sum(jnp.where(sel24, x24, 0.0), axis=1, keepdims=True)
        c0 = g[0:_B]
        c1 = g[_B:2 * _B]
        c2 = g[2 * _B:3 * _B]
        d0 = x0 - c0
        d1 = x1 - c1
        d2 = x2 - c2
        d = d0 * d0 + d1 * d1 + d2 * d2
        dist = jnp.minimum(dist_ref[...], d)
        dist_ref[...] = dist
        hit = col == i
        idx_ref[...] = jnp.where(hit, jnp.broadcast_to(far, (_B, _NC)), idx_ref[...])
        nx0_ref[...] = jnp.where(hit, jnp.broadcast_to(c0, (_B, _NC)), nx0_ref[...])
        nx1_ref[...] = jnp.where(hit, jnp.broadcast_to(c1, (_B, _NC)), nx1_ref[...])
        nx2_ref[...] = jnp.where(hit, jnp.broadcast_to(c2, (_B, _NC)), nx2_ref[...])
        m = jnp.max(dist, axis=1, keepdims=True)
        far_new = jnp.min(jnp.where(dist == m, lane, _N), axis=1, keepdims=True)
        return far_new

    jax.lax.fori_loop(0, _NC, body, far0_ref[...], unroll=2)


def _fps_call(xyz, far0):
    return pl.pallas_call(
        _fps_body,
        out_shape=(
            jax.ShapeDtypeStruct((_B, _NC), jnp.int32),
            jax.ShapeDtypeStruct((_B, _NC), jnp.float32),
            jax.ShapeDtypeStruct((_B, _NC), jnp.float32),
            jax.ShapeDtypeStruct((_B, _NC), jnp.float32),
        ),
        scratch_shapes=[pltpu.VMEM((_B, _N), jnp.float32)],
    )(xyz, far0)


def _bn(h, g, be):
    m = jnp.mean(h, axis=1, keepdims=True)
    v = jnp.mean((h - m) ** 2, axis=1, keepdims=True)
    return (h - m) / jnp.sqrt(v + _EPS) * g + be


def _lrelu(h):
    return jnp.where(h >= 0, h, 0.2 * h)


def _net_body(xt_ref, idx_ref, w1_ref, b1_ref, g1_ref, be1_ref, w2_ref, b2_ref,
              g2_ref, be2_ref, w3_ref, b3_ref, wq_ref, wk_ref, wv_ref, wo_ref,
              out_ref, f_scr):
    b = pl.program_id(0)

    @pl.when(b == 0)
    def _mlp():
        xt = xt_ref[...]
        h = jax.lax.dot_general(w1_ref[...], xt, (((1,), (0,)), ((), ())),
                                preferred_element_type=jnp.float32) + b1_ref[...]
        h = _lrelu(_bn(h, g1_ref[...], be1_ref[...]))
        h = jax.lax.dot_general(w2_ref[...], h, (((1,), (0,)), ((), ())),
                                preferred_element_type=jnp.float32) + b2_ref[...]
        h = _lrelu(_bn(h, g2_ref[...], be2_ref[...]))
        f_scr[...] = jax.lax.dot_general(w3_ref[...], h, (((1,), (0,)), ((), ())),
                                         preferred_element_type=jnp.float32) + b3_ref[...]

    fb = f_scr[:, pl.ds(b * _N, _N)]
    idxb = idx_ref[0]
    lane = jax.lax.broadcasted_iota(jnp.int32, (_NC, _N), 1)
    onehot = jnp.where(lane == idxb, 1.0, 0.0)
    cent = jax.lax.dot_general(onehot, fb, (((1,), (1,)), ((), ())),
                               preferred_element_type=jnp.float32)
    q = jax.lax.dot_general(cent, wq_ref[...], (((1,), (1,)), ((), ())),
                            preferred_element_type=jnp.float32)
    k = jax.lax.dot_general(wk_ref[...], fb, (((1,), (0,)), ((), ())),
                            preferred_element_type=jnp.float32)
    logits = jax.lax.dot_general(q, k, (((1,), (0,)), ((), ())),
                                 preferred_element_type=jnp.float32) * 0.125
    mx = jnp.max(logits, axis=1, keepdims=True)
    e = jnp.exp(logits - mx)
    probs = e / jnp.sum(e, axis=1, keepdims=True)
    t = jax.lax.dot_general(probs, fb, (((1,), (1,)), ((), ())),
                            preferred_element_type=jnp.float32)
    o = jax.lax.dot_general(t, wv_ref[...], (((1,), (1,)), ((), ())),
                            preferred_element_type=jnp.float32)
    y = jax.lax.dot_general(o, wo_ref[...], (((1,), (1,)), ((), ())),
                            preferred_element_type=jnp.float32)
    out_ref[0] = cent + y


def _net_call(xt, idx3, w1, b1, g1, be1, w2, b2, g2, be2, w3, b3, wq, wk, wv, wo):
    full2d = lambda a: pl.BlockSpec(a.shape, lambda b: (0,) * a.ndim)
    return pl.pallas_call(
        _net_body,
        grid=(_B,),
        in_specs=[
            full2d(xt),
            pl.BlockSpec((1, _NC, 1), lambda b: (b, 0, 0)),
        ] + [full2d(w) for w in (w1, b1, g1, be1, w2, b2, g2, be2, w3, b3,
                                 wq, wk, wv, wo)],
        out_specs=pl.BlockSpec((1, _NC, 64), lambda b: (b, 0, 0)),
        out_shape=jax.ShapeDtypeStruct((_B, _NC, 64), jnp.float32),
        scratch_shapes=[pltpu.VMEM((64, _B * _N), jnp.float32)],
        compiler_params=pltpu.CompilerParams(
            dimension_semantics=("arbitrary",)),
    )(xt, idx3, w1, b1, g1, be1, w2, b2, g2, be2, w3, b3, wq, wk, wv, wo)


def kernel(xyz, W1, b1, g1, be1, W2, b2, g2, be2, W3, b3, Wq, Wk, Wv, Wo):
    far0 = jax.random.randint(jax.random.key(42), (_B,), 0, _N,
                              dtype=jnp.int32).reshape(_B, 1)
    idx, nx0, nx1, nx2 = _fps_call(xyz, far0)
    new_xyz = jnp.stack([nx0, nx1, nx2], axis=1)

    xt = xyz.transpose(1, 0, 2).reshape(3, _B * _N)
    col = lambda a: a.reshape(-1, 1)
    out2t = _net_call(xt, idx.reshape(_B, _NC, 1), W1, col(b1), col(g1),
                      col(be1), W2, col(b2), col(g2), col(be2), W3, col(b3),
                      Wq, Wk, Wv, Wo)
    return (new_xyz, out2t.transpose(0, 2, 1))


# speedup vs baseline: 1.4083x; 1.0679x over previous
import jax
import jax.numpy as jnp
from jax.experimental import pallas as pl
from jax.experimental.pallas import tpu as pltpu

_B = 8
_N = 8192
_NC = 512
_EPS = 1e-5


def _fps_body(xyz_ref, far0_ref, idx_ref, nx0_ref, nx1_ref, nx2_ref, dist_ref):
    x0 = xyz_ref[:, 0, :]
    x1 = xyz_ref[:, 1, :]
    x2 = xyz_ref[:, 2, :]
    x24 = jnp.concatenate([x0, x1, x2], axis=0)
    lane = jax.lax.broadcasted_iota(jnp.int32, (_B, _N), 1)
    lane24 = jax.lax.broadcasted_iota(jnp.int32, (3 * _B, _N), 1)
    col = jax.lax.broadcasted_iota(jnp.int32, (_B, _NC), 1)
    dist_ref[...] = jnp.full((_B, _N), 1e10, jnp.float32)
    idx_ref[...] = jnp.zeros((_B, _NC), jnp.int32)
    nx0_ref[...] = jnp.zeros((_B, _NC), jnp.float32)
    nx1_ref[...] = jnp.zeros((_B, _NC), jnp.float32)
    nx2_ref[...] = jnp.zeros((_B, _NC), jnp.float32)

    def body(i, far):
        sel = lane == far
        sel24 = jnp.concatenate([sel, sel, sel], axis=0)
        g = jnp.sum(jnp.where(sel24, x24, 0.0), axis=1, keepdims=True)
        c0 = g[0:_B]
        c1 = g[_B:2 * _B]
        c2 = g[2 * _B:3 * _B]
        d0 = x0 - c0
        d1 = x1 - c1
        d2 = x2 - c2
        d = d0 * d0 + d1 * d1 + d2 * d2
        dist = jnp.minimum(dist_ref[...], d)
        dist_ref[...] = dist
        hit = col == i
        idx_ref[...] = jnp.where(hit, jnp.broadcast_to(far, (_B, _NC)), idx_ref[...])
        nx0_ref[...] = jnp.where(hit, jnp.broadcast_to(c0, (_B, _NC)), nx0_ref[...])
        nx1_ref[...] = jnp.where(hit, jnp.broadcast_to(c1, (_B, _NC)), nx1_ref[...])
        nx2_ref[...] = jnp.where(hit, jnp.broadcast_to(c2, (_B, _NC)), nx2_ref[...])
        far_new = jnp.argmax(dist, axis=1).astype(jnp.int32)[:, None]
        return far_new

    jax.lax.fori_loop(0, _NC, body, far0_ref[...], unroll=2)


def _fps_call(xyz, far0):
    return pl.pallas_call(
        _fps_body,
        out_shape=(
            jax.ShapeDtypeStruct((_B, _NC), jnp.int32),
            jax.ShapeDtypeStruct((_B, _NC), jnp.float32),
            jax.ShapeDtypeStruct((_B, _NC), jnp.float32),
            jax.ShapeDtypeStruct((_B, _NC), jnp.float32),
        ),
        scratch_shapes=[pltpu.VMEM((_B, _N), jnp.float32)],
    )(xyz, far0)


def _bn(h, g, be):
    m = jnp.mean(h, axis=1, keepdims=True)
    v = jnp.mean((h - m) ** 2, axis=1, keepdims=True)
    return (h - m) / jnp.sqrt(v + _EPS) * g + be


def _lrelu(h):
    return jnp.where(h >= 0, h, 0.2 * h)


def _net_body(xt_ref, idx_ref, w1_ref, b1_ref, g1_ref, be1_ref, w2_ref, b2_ref,
              g2_ref, be2_ref, w3_ref, b3_ref, wq_ref, wk_ref, wv_ref, wo_ref,
              out_ref, f_scr):
    b = pl.program_id(0)

    @pl.when(b == 0)
    def _mlp():
        xt = xt_ref[...]
        h = jax.lax.dot_general(w1_ref[...], xt, (((1,), (0,)), ((), ())),
                                preferred_element_type=jnp.float32) + b1_ref[...]
        h = _lrelu(_bn(h, g1_ref[...], be1_ref[...]))
        h = jax.lax.dot_general(w2_ref[...], h, (((1,), (0,)), ((), ())),
                                preferred_element_type=jnp.float32) + b2_ref[...]
        h = _lrelu(_bn(h, g2_ref[...], be2_ref[...]))
        f_scr[...] = jax.lax.dot_general(w3_ref[...], h, (((1,), (0,)), ((), ())),
                                         preferred_element_type=jnp.float32) + b3_ref[...]

    fb = f_scr[:, pl.ds(b * _N, _N)]
    idxb = idx_ref[0]
    lane = jax.lax.broadcasted_iota(jnp.int32, (_NC, _N), 1)
    onehot = jnp.where(lane == idxb, 1.0, 0.0)
    cent = jax.lax.dot_general(onehot, fb, (((1,), (1,)), ((), ())),
                               preferred_element_type=jnp.float32)
    q = jax.lax.dot_general(cent, wq_ref[...], (((1,), (1,)), ((), ())),
                            preferred_element_type=jnp.float32)
    k = jax.lax.dot_general(wk_ref[...], fb, (((1,), (0,)), ((), ())),
                            preferred_element_type=jnp.float32)
    logits = jax.lax.dot_general(q, k, (((1,), (0,)), ((), ())),
                                 preferred_element_type=jnp.float32) * 0.125
    mx = jnp.max(logits, axis=1, keepdims=True)
    e = jnp.exp(logits - mx)
    probs = e / jnp.sum(e, axis=1, keepdims=True)
    t = jax.lax.dot_general(probs, fb, (((1,), (1,)), ((), ())),
                            preferred_element_type=jnp.float32)
    o = jax.lax.dot_general(t, wv_ref[...], (((1,), (1,)), ((), ())),
                            preferred_element_type=jnp.float32)
    y = jax.lax.dot_general(o, wo_ref[...], (((1,), (1,)), ((), ())),
                            preferred_element_type=jnp.float32)
    out_ref[0] = cent + y


def _net_call(xt, idx3, w1, b1, g1, be1, w2, b2, g2, be2, w3, b3, wq, wk, wv, wo):
    full2d = lambda a: pl.BlockSpec(a.shape, lambda b: (0,) * a.ndim)
    return pl.pallas_call(
        _net_body,
        grid=(_B,),
        in_specs=[
            full2d(xt),
            pl.BlockSpec((1, _NC, 1), lambda b: (b, 0, 0)),
        ] + [full2d(w) for w in (w1, b1, g1, be1, w2, b2, g2, be2, w3, b3,
                                 wq, wk, wv, wo)],
        out_specs=pl.BlockSpec((1, _NC, 64), lambda b: (b, 0, 0)),
        out_shape=jax.ShapeDtypeStruct((_B, _NC, 64), jnp.float32),
        scratch_shapes=[pltpu.VMEM((64, _B * _N), jnp.float32)],
        compiler_params=pltpu.CompilerParams(
            dimension_semantics=("arbitrary",)),
    )(xt, idx3, w1, b1, g1, be1, w2, b2, g2, be2, w3, b3, wq, wk, wv, wo)


def kernel(xyz, W1, b1, g1, be1, W2, b2, g2, be2, W3, b3, Wq, Wk, Wv, Wo):
    far0 = jax.random.randint(jax.random.key(42), (_B,), 0, _N,
                              dtype=jnp.int32).reshape(_B, 1)
    idx, nx0, nx1, nx2 = _fps_call(xyz, far0)
    new_xyz = jnp.stack([nx0, nx1, nx2], axis=1)

    xt = xyz.transpose(1, 0, 2).reshape(3, _B * _N)
    col = lambda a: a.reshape(-1, 1)
    out2t = _net_call(xt, idx.reshape(_B, _NC, 1), W1, col(b1), col(g1),
                      col(be1), W2, col(b2), col(g2), col(be2), W3, col(b3),
                      Wq, Wk, Wv, Wo)
    return (new_xyz, out2t.transpose(0, 2, 1))
